# baseline (speedup 1.0000x reference)
import functools

import jax
import jax.numpy as jnp
from jax import lax
from jax.experimental import pallas as pl
from jax.experimental.pallas import tpu as pltpu
from jax.experimental.pallas import tpu_sc as plsc

_B = 16384
_D = 16
_H = 16
_NC = 2
_NS = 16
_NW = _NC * _NS
_BPW = _B // _NW
_CHUNK = 128
_NCH = _BPW // _CHUNK
_G = _BPW // 16


def _sc_body(uidx_hbm, eidx_hbm, utab_hbm, etab_hbm, w1s_hbm, b1s_hbm,
             w2s_hbm, b2s_hbm, out_hbm,
             uidx_v, eidx_v, urows_v, erows_v, w1s_v, b1s_v, w2s_v, b2s_v,
             out_v, sem_u, sem_e):
    wid = lax.axis_index("s") * _NC + lax.axis_index("c")
    base = wid * _BPW
    pltpu.sync_copy(uidx_hbm.at[pl.ds(base, _BPW)], uidx_v)
    pltpu.sync_copy(eidx_hbm.at[pl.ds(base, _BPW)], eidx_v)
    copies = []
    for j in range(_NCH):
        sl = pl.ds(j * _CHUNK, _CHUNK)
        copies.append(pltpu.async_copy(
            utab_hbm.at[uidx_v.at[sl]], urows_v.at[sl], sem_u))
        copies.append(pltpu.async_copy(
            etab_hbm.at[eidx_v.at[sl]], erows_v.at[sl], sem_e))
    pltpu.sync_copy(w1s_hbm, w1s_v)
    pltpu.sync_copy(b1s_hbm, b1s_v)
    pltpu.sync_copy(w2s_hbm, w2s_v)
    pltpu.sync_copy(b2s_hbm, b2s_v)
    for c in copies:
        c.wait()

    def group(g, _):
        rows = g * 16 + lax.iota(jnp.int32, 16)
        xs = []
        for k in range(_D):
            col = jnp.full((16,), k, jnp.int32)
            xs.append(plsc.load_gather(urows_v, [rows, col]))
        for k in range(_D):
            col = jnp.full((16,), k, jnp.int32)
            xs.append(plsc.load_gather(erows_v, [rows, col]))
        o = b2s_v[...]
        for j in range(_H):
            acc = b1s_v[j]
            for k in range(2 * _D):
                acc = acc + w1s_v[j * 2 * _D + k] * xs[k]
            o = o + w2s_v[j] * jnp.maximum(acc, 0.0)
        out_v[pl.ds(g * 16, 16)] = 1.0 / (1.0 + jnp.exp(-o))
        return 0

    lax.fori_loop(0, _G, group, 0)
    pltpu.sync_copy(out_v, out_hbm.at[pl.ds(base, _BPW)])


_sc_ncf = functools.partial(
    pl.kernel,
    out_type=jax.ShapeDtypeStruct((_B,), jnp.float32),
    mesh=plsc.VectorSubcoreMesh(core_axis_name="c", subcore_axis_name="s"),
    compiler_params=pltpu.CompilerParams(use_tc_tiling_on_sc=False,
                                         needs_layout_passes=False),
    scratch_types=[
        pltpu.VMEM((_BPW,), jnp.int32),
        pltpu.VMEM((_BPW,), jnp.int32),
        pltpu.VMEM((_BPW, _D), jnp.float32),
        pltpu.VMEM((_BPW, _D), jnp.float32),
        pltpu.VMEM((_H * 2 * _D, 16), jnp.float32),
        pltpu.VMEM((_H, 16), jnp.float32),
        pltpu.VMEM((_H, 16), jnp.float32),
        pltpu.VMEM((16,), jnp.float32),
        pltpu.VMEM((_BPW,), jnp.float32),
        pltpu.SemaphoreType.DMA,
        pltpu.SemaphoreType.DMA,
    ],
)(_sc_body)


def kernel(user, event, user_emb, event_emb, W1, b1, W2, b2):
    w1s = jnp.broadcast_to(W1.reshape(_H * 2 * _D, 1), (_H * 2 * _D, 16))
    b1s = jnp.broadcast_to(b1.reshape(_H, 1), (_H, 16))
    w2s = jnp.broadcast_to(W2.reshape(_H, 1), (_H, 16))
    b2s = jnp.broadcast_to(b2.reshape(1), (16,))
    out = _sc_ncf(user, event, user_emb, event_emb, w1s, b1s, w2s, b2s)
    return out.reshape(_B, 1)
